# baseline (speedup 1.0000x reference)
import functools

import jax
import jax.numpy as jnp
from jax import lax
from jax.experimental import pallas as pl
from jax.experimental.pallas import tpu as pltpu
from jax.experimental.pallas import tpu_sc as plsc

D_EMB = 64
MAXN = 10.0
L = 16
NCORE = 2
NSUB = 16
NWORK = NCORE * NSUB
BAG = 50
GBAGS = 8
GROWS = GBAGS * BAG
NSTREAM = 5
SPG = GROWS // NSTREAM
RSQRT_MAGIC = 0x5F3759DF


def _make_tc_transpose(n, blk_cols=512):
    grid = (n + blk_cols - 1) // blk_cols
    half = blk_cols // 2

    def body(x_ref, o_ref):
        x = x_ref[...]
        eye = jnp.eye(D_EMB, dtype=jnp.float32)
        xt = lax.dot_general(x, eye, (((0,), (0,)), ((), ())),
                             precision=lax.Precision.HIGHEST,
                             preferred_element_type=jnp.float32)
        o_ref[:, 0:D_EMB] = xt[0:half]
        o_ref[:, D_EMB:2 * D_EMB] = xt[half:blk_cols]

    return pl.pallas_call(
        body,
        grid=(grid,),
        in_specs=[pl.BlockSpec((D_EMB, blk_cols), lambda j: (0, j))],
        out_specs=pl.BlockSpec((half, 2 * D_EMB), lambda j: (j, 0)),
        out_shape=jax.ShapeDtypeStruct((grid * half, 2 * D_EMB), jnp.float32),
    )


def _rsqrt(x):
    i = lax.bitcast_convert_type(x, jnp.int32)
    y = lax.bitcast_convert_type(
        jnp.int32(RSQRT_MAGIC) - lax.shift_right_logical(i, 1), jnp.float32)
    for _ in range(2):
        y = y * (1.5 - 0.5 * x * y * y)
    return y


def _make_gather(batch):
    bags_per_worker = batch // NWORK
    ngrp = bags_per_worker // GBAGS
    rows_per_worker = ngrp * GROWS

    mesh = plsc.VectorSubcoreMesh(core_axis_name="c", subcore_axis_name="s")

    @functools.partial(
        pl.kernel,
        out_type=jax.ShapeDtypeStruct((batch, D_EMB), jnp.float32),
        mesh=mesh,
        scratch_types=[
            pltpu.VMEM((rows_per_worker,), jnp.int32),
            pltpu.VMEM((GROWS,), jnp.int32),
            pltpu.VMEM((GROWS,), jnp.int32),
            pltpu.VMEM((2, GROWS, 2 * D_EMB), jnp.float32),
            pltpu.VMEM((L * L,), jnp.float32),
            pltpu.VMEM((GROWS,), jnp.float32),
            pltpu.VMEM((GBAGS, D_EMB), jnp.float32),
            pltpu.SemaphoreType.DMA,
            pltpu.SemaphoreType.DMA,
        ],
        compiler_params=pltpu.CompilerParams(needs_layout_passes=False,
                                             use_tc_tiling_on_sc=True),
    )
    def sc_gather(idx_hbm, tab_hbm, res_hbm,
                  idx_v, sidx0, sidx1, rowbuf, tbuf, scale_v, acc_v,
                  sem0, sem1):
        wid = lax.axis_index("s") * NCORE + lax.axis_index("c")
        sems = (sem0, sem1)
        sidxs = (sidx0, sidx1)
        iota = lax.iota(jnp.int32, L)
        zero = jnp.zeros((L,), jnp.float32)

        pltpu.sync_copy(
            idx_hbm.at[pl.ds(wid * rows_per_worker, rows_per_worker)], idx_v)

        def issue(g, b):
            base = g * GROWS

            def shift_blk(k, carry):
                iv = idx_v[pl.ds(base + k * L, L)]
                vr = lax.shift_left(lax.shift_right_logical(iv, 9), 8) | (
                    iv & 255)
                sidxs[b][pl.ds(k * L, L)] = vr
                return carry

            lax.fori_loop(0, GROWS // L, shift_blk, 0)
            for j in range(NSTREAM):
                pltpu.async_copy(
                    tab_hbm.at[sidxs[b].at[pl.ds(j * SPG, SPG)]],
                    rowbuf.at[b, pl.ds(j * SPG, SPG)],
                    sems[b])

        def wait_group(b):
            pltpu.make_async_copy(tab_hbm.at[pl.ds(0, GROWS)],
                                  rowbuf.at[b], sems[b]).wait()

        def compute(g, b):
            base = g * GROWS

            def blk_body(blk, carry):
                r0 = blk * L
                ivec = idx_v[pl.ds(base + r0, L)]
                for j in range(L):
                    half = (lax.shift_right_logical(ivec[j], 8) & 1) * D_EMB
                    n2v = None
                    for c in range(4):
                        ch = rowbuf[b, r0 + j, pl.ds(half + c * L, L)]
                        n2v = ch * ch if n2v is None else n2v + ch * ch
                    plsc.store_scatter(tbuf, [iota * L + j], n2v)
                n2 = tbuf[pl.ds(0, L)]
                for lrow in range(1, L):
                    n2 = n2 + tbuf[pl.ds(lrow * L, L)]
                scale = jnp.minimum(1.0, MAXN * _rsqrt(n2))
                scale_v[pl.ds(r0, L)] = scale
                return carry

            lax.fori_loop(0, GROWS // L, blk_body, 0)

            for bag in range(GBAGS):
                for c in range(4):
                    acc_v[bag, pl.ds(c * L, L)] = zero

            def acc_body(blk, carry):
                r0 = blk * L
                svec = scale_v[pl.ds(r0, L)]
                ivec = idx_v[pl.ds(base + r0, L)]
                for j in range(L):
                    half = (lax.shift_right_logical(ivec[j], 8) & 1) * D_EMB
                    s = jnp.take(svec, jnp.full((L,), j, jnp.int32))
                    bag = (r0 + j) // BAG
                    for c in range(4):
                        ch = rowbuf[b, r0 + j, pl.ds(half + c * L, L)]
                        plsc.addupdate(acc_v.at[bag, pl.ds(c * L, L)], s * ch)
                return carry

            lax.fori_loop(0, GROWS // L, acc_body, 0)

            base_row = wid * bags_per_worker + g * GBAGS
            pltpu.sync_copy(acc_v, res_hbm.at[pl.ds(base_row, GBAGS)])

        issue(0, 0)
        issue(1, 1)

        def pair_body(i, carry):
            for b in range(2):
                g = i * 2 + b
                wait_group(b)
                compute(g, b)

                @pl.when(g + 2 < ngrp)
                def _():
                    issue(g + 2, b)
            return carry

        lax.fori_loop(0, ngrp // 2, pair_body, 0)

    return sc_gather


def kernel(input, output, input_table, output_table):
    batch = input.shape[0]
    gather = _make_gather(batch)
    n_in = input_table.shape[0]
    n_out = output_table.shape[0]

    in_tab = _make_tc_transpose(n_in)(input_table.T)
    out_tab = _make_tc_transpose(n_out)(output_table.T)

    in_res = gather(input.reshape(batch * BAG), in_tab)
    out_res = gather(output.reshape(batch * BAG), out_tab)
    return (in_res, out_res)


# speedup vs baseline: 2.5844x; 2.5844x over previous
import functools

import jax
import jax.numpy as jnp
from jax import lax
from jax.experimental import pallas as pl
from jax.experimental.pallas import tpu as pltpu
from jax.experimental.pallas import tpu_sc as plsc

D_EMB = 64
MAXN = 10.0
L = 16
NCORE = 2
NSUB = 16
NWORK = NCORE * NSUB
BAG = 50
GBAGS = 8
GROWS = GBAGS * BAG
NSTREAM = 5
SPG = GROWS // NSTREAM
RSQRT_MAGIC = 0x5F3759DF


BLK_COLS = 4096
VSHIFT = 12
HSHIFT = 11
VMASK = (1 << HSHIFT) - 1


def _make_tc_transpose(n, blk_cols=BLK_COLS):
    grid = (n + blk_cols - 1) // blk_cols
    half = blk_cols // 2

    def body(x_ref, o_ref):
        x = x_ref[...]
        eye = jnp.eye(D_EMB, dtype=jnp.float32)
        xt = lax.dot_general(x, eye, (((0,), (0,)), ((), ())),
                             precision=lax.Precision.HIGHEST,
                             preferred_element_type=jnp.float32)
        o_ref[:, 0:D_EMB] = xt[0:half]
        o_ref[:, D_EMB:2 * D_EMB] = xt[half:blk_cols]

    return pl.pallas_call(
        body,
        grid=(grid,),
        in_specs=[pl.BlockSpec((D_EMB, blk_cols), lambda j: (0, j))],
        out_specs=pl.BlockSpec((half, 2 * D_EMB), lambda j: (j, 0)),
        out_shape=jax.ShapeDtypeStruct((grid * half, 2 * D_EMB), jnp.float32),
    )


def _rsqrt(x):
    i = lax.bitcast_convert_type(x, jnp.int32)
    y = lax.bitcast_convert_type(
        jnp.int32(RSQRT_MAGIC) - lax.shift_right_logical(i, 1), jnp.float32)
    for _ in range(2):
        y = y * (1.5 - 0.5 * x * y * y)
    return y


def _make_gather(batch):
    bags_per_worker = batch // NWORK
    ngrp = bags_per_worker // GBAGS
    rows_per_worker = ngrp * GROWS

    mesh = plsc.VectorSubcoreMesh(core_axis_name="c", subcore_axis_name="s")

    @functools.partial(
        pl.kernel,
        out_type=jax.ShapeDtypeStruct((batch, D_EMB), jnp.float32),
        mesh=mesh,
        scratch_types=[
            pltpu.VMEM((rows_per_worker,), jnp.int32),
            pltpu.VMEM((GROWS,), jnp.int32),
            pltpu.VMEM((GROWS,), jnp.int32),
            pltpu.VMEM((2, GROWS, 2 * D_EMB), jnp.float32),
            pltpu.VMEM((L * L,), jnp.float32),
            pltpu.VMEM((GROWS,), jnp.float32),
            pltpu.VMEM((GBAGS, D_EMB), jnp.float32),
            pltpu.SemaphoreType.DMA,
            pltpu.SemaphoreType.DMA,
        ],
        compiler_params=pltpu.CompilerParams(needs_layout_passes=False,
                                             use_tc_tiling_on_sc=True),
    )
    def sc_gather(idx_hbm, tab_hbm, res_hbm,
                  idx_v, sidx0, sidx1, rowbuf, tbuf, scale_v, acc_v,
                  sem0, sem1):
        wid = lax.axis_index("s") * NCORE + lax.axis_index("c")
        sems = (sem0, sem1)
        sidxs = (sidx0, sidx1)
        iota = lax.iota(jnp.int32, L)
        zero = jnp.zeros((L,), jnp.float32)

        pltpu.sync_copy(
            idx_hbm.at[pl.ds(wid * rows_per_worker, rows_per_worker)], idx_v)

        def issue(g, b):
            base = g * GROWS

            def shift_blk(k, carry):
                iv = idx_v[pl.ds(base + k * L, L)]
                vr = lax.shift_left(
                    lax.shift_right_logical(iv, VSHIFT), HSHIFT) | (
                        iv & VMASK)
                sidxs[b][pl.ds(k * L, L)] = vr
                return carry

            lax.fori_loop(0, GROWS // L, shift_blk, 0)
            for j in range(NSTREAM):
                pltpu.async_copy(
                    tab_hbm.at[sidxs[b].at[pl.ds(j * SPG, SPG)]],
                    rowbuf.at[b, pl.ds(j * SPG, SPG)],
                    sems[b])

        def wait_group(b):
            pltpu.make_async_copy(tab_hbm.at[pl.ds(0, GROWS)],
                                  rowbuf.at[b], sems[b]).wait()

        def compute(g, b):
            base = g * GROWS

            def blk_body(blk, carry):
                r0 = blk * L
                ivec = idx_v[pl.ds(base + r0, L)]
                for j in range(L):
                    half = (lax.shift_right_logical(ivec[j], HSHIFT) & 1) * D_EMB
                    n2v = None
                    for c in range(4):
                        ch = rowbuf[b, r0 + j, pl.ds(half + c * L, L)]
                        n2v = ch * ch if n2v is None else n2v + ch * ch
                    plsc.store_scatter(tbuf, [iota * L + j], n2v)
                n2 = tbuf[pl.ds(0, L)]
                for lrow in range(1, L):
                    n2 = n2 + tbuf[pl.ds(lrow * L, L)]
                scale = jnp.minimum(1.0, MAXN * _rsqrt(n2))
                scale_v[pl.ds(r0, L)] = scale
                return carry

            lax.fori_loop(0, GROWS // L, blk_body, 0)

            for bag in range(GBAGS):
                for c in range(4):
                    acc_v[bag, pl.ds(c * L, L)] = zero

            def acc_body(blk, carry):
                r0 = blk * L
                svec = scale_v[pl.ds(r0, L)]
                ivec = idx_v[pl.ds(base + r0, L)]
                for j in range(L):
                    half = (lax.shift_right_logical(ivec[j], HSHIFT) & 1) * D_EMB
                    s = jnp.take(svec, jnp.full((L,), j, jnp.int32))
                    bag = (r0 + j) // BAG
                    for c in range(4):
                        ch = rowbuf[b, r0 + j, pl.ds(half + c * L, L)]
                        plsc.addupdate(acc_v.at[bag, pl.ds(c * L, L)], s * ch)
                return carry

            lax.fori_loop(0, GROWS // L, acc_body, 0)

            base_row = wid * bags_per_worker + g * GBAGS
            pltpu.sync_copy(acc_v, res_hbm.at[pl.ds(base_row, GBAGS)])

        issue(0, 0)
        issue(1, 1)

        def pair_body(i, carry):
            for b in range(2):
                g = i * 2 + b
                wait_group(b)
                compute(g, b)

                @pl.when(g + 2 < ngrp)
                def _():
                    issue(g + 2, b)
            return carry

        lax.fori_loop(0, ngrp // 2, pair_body, 0)

    return sc_gather


def kernel(input, output, input_table, output_table):
    batch = input.shape[0]
    gather = _make_gather(batch)
    n_in = input_table.shape[0]
    n_out = output_table.shape[0]

    in_tab = _make_tc_transpose(n_in)(input_table.T)
    out_tab = _make_tc_transpose(n_out)(output_table.T)

    in_res = gather(input.reshape(batch * BAG), in_tab)
    out_res = gather(output.reshape(batch * BAG), out_tab)
    return (in_res, out_res)


# speedup vs baseline: 2.7689x; 1.0714x over previous
import functools

import jax
import jax.numpy as jnp
from jax import lax
from jax.experimental import pallas as pl
from jax.experimental.pallas import tpu as pltpu
from jax.experimental.pallas import tpu_sc as plsc

D_EMB = 64
MAXN = 10.0
L = 16
NCORE = 2
NSUB = 16
NWORK = NCORE * NSUB
BAG = 50
GBAGS = 8
GROWS = GBAGS * BAG
NSTREAM = 5
SPG = GROWS // NSTREAM
RSQRT_MAGIC = 0x5F3759DF


BLK_COLS = 8192
VSHIFT = 13
HSHIFT = 12
VMASK = (1 << HSHIFT) - 1


def _make_tc_transpose(n, blk_cols=BLK_COLS):
    grid = (n + blk_cols - 1) // blk_cols
    half = blk_cols // 2

    def body(x_ref, o_ref):
        x = x_ref[...]
        eye = jnp.eye(D_EMB, dtype=jnp.float32)
        xt = lax.dot_general(x, eye, (((0,), (0,)), ((), ())),
                             precision=lax.Precision.HIGHEST,
                             preferred_element_type=jnp.float32)
        o_ref[...] = jnp.concatenate([xt[0:half], xt[half:blk_cols]], axis=1)

    return pl.pallas_call(
        body,
        grid=(grid,),
        in_specs=[pl.BlockSpec((D_EMB, blk_cols), lambda j: (0, j))],
        out_specs=pl.BlockSpec((half, 2 * D_EMB), lambda j: (j, 0)),
        out_shape=jax.ShapeDtypeStruct((grid * half, 2 * D_EMB), jnp.float32),
    )


def _rsqrt(x):
    i = lax.bitcast_convert_type(x, jnp.int32)
    y = lax.bitcast_convert_type(
        jnp.int32(RSQRT_MAGIC) - lax.shift_right_logical(i, 1), jnp.float32)
    for _ in range(2):
        y = y * (1.5 - 0.5 * x * y * y)
    return y


def _make_gather(batch):
    bags_per_worker = batch // NWORK
    ngrp = bags_per_worker // GBAGS
    rows_per_worker = ngrp * GROWS

    mesh = plsc.VectorSubcoreMesh(core_axis_name="c", subcore_axis_name="s")

    @functools.partial(
        pl.kernel,
        out_type=jax.ShapeDtypeStruct((batch, D_EMB), jnp.float32),
        mesh=mesh,
        scratch_types=[
            pltpu.VMEM((rows_per_worker,), jnp.int32),
            pltpu.VMEM((GROWS,), jnp.int32),
            pltpu.VMEM((GROWS,), jnp.int32),
            pltpu.VMEM((2, GROWS, 2 * D_EMB), jnp.float32),
            pltpu.VMEM((L * L,), jnp.float32),
            pltpu.VMEM((GROWS,), jnp.float32),
            pltpu.VMEM((GBAGS, D_EMB), jnp.float32),
            pltpu.SemaphoreType.DMA,
            pltpu.SemaphoreType.DMA,
        ],
        compiler_params=pltpu.CompilerParams(needs_layout_passes=False,
                                             use_tc_tiling_on_sc=True),
    )
    def sc_gather(idx_hbm, tab_hbm, res_hbm,
                  idx_v, sidx0, sidx1, rowbuf, tbuf, scale_v, acc_v,
                  sem0, sem1):
        wid = lax.axis_index("s") * NCORE + lax.axis_index("c")
        sems = (sem0, sem1)
        sidxs = (sidx0, sidx1)
        iota = lax.iota(jnp.int32, L)
        zero = jnp.zeros((L,), jnp.float32)

        pltpu.sync_copy(
            idx_hbm.at[pl.ds(wid * rows_per_worker, rows_per_worker)], idx_v)

        def issue(g, b):
            base = g * GROWS

            def shift_blk(k, carry):
                iv = idx_v[pl.ds(base + k * L, L)]
                vr = lax.shift_left(
                    lax.shift_right_logical(iv, VSHIFT), HSHIFT) | (
                        iv & VMASK)
                sidxs[b][pl.ds(k * L, L)] = vr
                return carry

            lax.fori_loop(0, GROWS // L, shift_blk, 0)
            for j in range(NSTREAM):
                pltpu.async_copy(
                    tab_hbm.at[sidxs[b].at[pl.ds(j * SPG, SPG)]],
                    rowbuf.at[b, pl.ds(j * SPG, SPG)],
                    sems[b])

        def wait_group(b):
            pltpu.make_async_copy(tab_hbm.at[pl.ds(0, GROWS)],
                                  rowbuf.at[b], sems[b]).wait()

        def compute(g, b):
            base = g * GROWS

            def blk_body(blk, carry):
                r0 = blk * L
                ivec = idx_v[pl.ds(base + r0, L)]
                for j in range(L):
                    half = (lax.shift_right_logical(ivec[j], HSHIFT) & 1) * D_EMB
                    n2v = None
                    for c in range(4):
                        ch = rowbuf[b, r0 + j, pl.ds(half + c * L, L)]
                        n2v = ch * ch if n2v is None else n2v + ch * ch
                    plsc.store_scatter(tbuf, [iota * L + j], n2v)
                n2 = tbuf[pl.ds(0, L)]
                for lrow in range(1, L):
                    n2 = n2 + tbuf[pl.ds(lrow * L, L)]
                scale = jnp.minimum(1.0, MAXN * _rsqrt(n2))
                scale_v[pl.ds(r0, L)] = scale
                return carry

            lax.fori_loop(0, GROWS // L, blk_body, 0)

            for bag in range(GBAGS):
                for c in range(4):
                    acc_v[bag, pl.ds(c * L, L)] = zero

            def acc_body(blk, carry):
                r0 = blk * L
                svec = scale_v[pl.ds(r0, L)]
                ivec = idx_v[pl.ds(base + r0, L)]
                for j in range(L):
                    half = (lax.shift_right_logical(ivec[j], HSHIFT) & 1) * D_EMB
                    s = jnp.take(svec, jnp.full((L,), j, jnp.int32))
                    bag = (r0 + j) // BAG
                    for c in range(4):
                        ch = rowbuf[b, r0 + j, pl.ds(half + c * L, L)]
                        plsc.addupdate(acc_v.at[bag, pl.ds(c * L, L)], s * ch)
                return carry

            lax.fori_loop(0, GROWS // L, acc_body, 0)

            base_row = wid * bags_per_worker + g * GBAGS
            pltpu.sync_copy(acc_v, res_hbm.at[pl.ds(base_row, GBAGS)])

        issue(0, 0)
        issue(1, 1)

        def pair_body(i, carry):
            for b in range(2):
                g = i * 2 + b
                wait_group(b)
                compute(g, b)

                @pl.when(g + 2 < ngrp)
                def _():
                    issue(g + 2, b)
            return carry

        lax.fori_loop(0, ngrp // 2, pair_body, 0)

    return sc_gather


def kernel(input, output, input_table, output_table):
    batch = input.shape[0]
    gather = _make_gather(batch)
    n_in = input_table.shape[0]
    n_out = output_table.shape[0]

    in_tab = _make_tc_transpose(n_in)(input_table.T)
    out_tab = _make_tc_transpose(n_out)(output_table.T)

    in_res = gather(input.reshape(batch * BAG), in_tab)
    out_res = gather(output.reshape(batch * BAG), out_tab)
    return (in_res, out_res)


# speedup vs baseline: 4.0655x; 1.4683x over previous
import functools

import jax
import jax.numpy as jnp
from jax import lax
from jax.experimental import pallas as pl
from jax.experimental.pallas import tpu as pltpu
from jax.experimental.pallas import tpu_sc as plsc

D_EMB = 64
MAXN = 10.0
L = 16
NCORE = 2
NSUB = 16
NWORK = NCORE * NSUB
BAG = 50
GBAGS = 8
GROWS = GBAGS * BAG
NSTREAM = 5
SPG = GROWS // NSTREAM
RSQRT_MAGIC = 0x5F3759DF


BLK_COLS = 8192
VSHIFT = 13
HSHIFT = 12
VMASK = (1 << HSHIFT) - 1


def _make_tc_transpose(n, blk_cols=BLK_COLS):
    grid = (n + blk_cols - 1) // blk_cols
    half = blk_cols // 2

    def body(x_ref, o_ref):
        x = x_ref[...]
        eye = jnp.eye(D_EMB, dtype=jnp.float32)
        xt = lax.dot_general(x, eye, (((0,), (0,)), ((), ())),
                             precision=lax.Precision.DEFAULT,
                             preferred_element_type=jnp.float32)
        o_ref[...] = jnp.concatenate([xt[0:half], xt[half:blk_cols]], axis=1)

    return pl.pallas_call(
        body,
        grid=(grid,),
        in_specs=[pl.BlockSpec((D_EMB, blk_cols), lambda j: (0, j))],
        out_specs=pl.BlockSpec((half, 2 * D_EMB), lambda j: (j, 0)),
        out_shape=jax.ShapeDtypeStruct((grid * half, 2 * D_EMB), jnp.float32),
    )


def _rsqrt(x):
    i = lax.bitcast_convert_type(x, jnp.int32)
    y = lax.bitcast_convert_type(
        jnp.int32(RSQRT_MAGIC) - lax.shift_right_logical(i, 1), jnp.float32)
    for _ in range(2):
        y = y * (1.5 - 0.5 * x * y * y)
    return y


def _make_gather(batch):
    bags_per_worker = batch // NWORK
    ngrp = bags_per_worker // GBAGS
    rows_per_worker = ngrp * GROWS

    mesh = plsc.VectorSubcoreMesh(core_axis_name="c", subcore_axis_name="s")

    @functools.partial(
        pl.kernel,
        out_type=jax.ShapeDtypeStruct((batch, D_EMB), jnp.float32),
        mesh=mesh,
        scratch_types=[
            pltpu.VMEM((rows_per_worker,), jnp.int32),
            pltpu.VMEM((GROWS,), jnp.int32),
            pltpu.VMEM((GROWS,), jnp.int32),
            pltpu.VMEM((2, GROWS, 2 * D_EMB), jnp.float32),
            pltpu.VMEM((L * L,), jnp.float32),
            pltpu.VMEM((GROWS,), jnp.float32),
            pltpu.VMEM((GBAGS, D_EMB), jnp.float32),
            pltpu.SemaphoreType.DMA,
            pltpu.SemaphoreType.DMA,
        ],
        compiler_params=pltpu.CompilerParams(needs_layout_passes=False,
                                             use_tc_tiling_on_sc=True),
    )
    def sc_gather(idx_hbm, tab_hbm, res_hbm,
                  idx_v, sidx0, sidx1, rowbuf, tbuf, scale_v, acc_v,
                  sem0, sem1):
        wid = lax.axis_index("s") * NCORE + lax.axis_index("c")
        sems = (sem0, sem1)
        sidxs = (sidx0, sidx1)
        iota = lax.iota(jnp.int32, L)
        zero = jnp.zeros((L,), jnp.float32)

        pltpu.sync_copy(
            idx_hbm.at[pl.ds(wid * rows_per_worker, rows_per_worker)], idx_v)

        def issue(g, b):
            base = g * GROWS

            def shift_blk(k, carry):
                iv = idx_v[pl.ds(base + k * L, L)]
                vr = lax.shift_left(
                    lax.shift_right_logical(iv, VSHIFT), HSHIFT) | (
                        iv & VMASK)
                sidxs[b][pl.ds(k * L, L)] = vr
                return carry

            lax.fori_loop(0, GROWS // L, shift_blk, 0)
            for j in range(NSTREAM):
                pltpu.async_copy(
                    tab_hbm.at[sidxs[b].at[pl.ds(j * SPG, SPG)]],
                    rowbuf.at[b, pl.ds(j * SPG, SPG)],
                    sems[b])

        def wait_group(b):
            pltpu.make_async_copy(tab_hbm.at[pl.ds(0, GROWS)],
                                  rowbuf.at[b], sems[b]).wait()

        def compute(g, b):
            base = g * GROWS

            def blk_body(blk, carry):
                r0 = blk * L
                ivec = idx_v[pl.ds(base + r0, L)]
                for j in range(L):
                    half = (lax.shift_right_logical(ivec[j], HSHIFT) & 1) * D_EMB
                    n2v = None
                    for c in range(4):
                        ch = rowbuf[b, r0 + j, pl.ds(half + c * L, L)]
                        n2v = ch * ch if n2v is None else n2v + ch * ch
                    plsc.store_scatter(tbuf, [iota * L + j], n2v)
                n2 = tbuf[pl.ds(0, L)]
                for lrow in range(1, L):
                    n2 = n2 + tbuf[pl.ds(lrow * L, L)]
                scale = jnp.minimum(1.0, MAXN * _rsqrt(n2))
                scale_v[pl.ds(r0, L)] = scale
                return carry

            lax.fori_loop(0, GROWS // L, blk_body, 0)

            for bag in range(GBAGS):
                for c in range(4):
                    acc_v[bag, pl.ds(c * L, L)] = zero

            def acc_body(blk, carry):
                r0 = blk * L
                svec = scale_v[pl.ds(r0, L)]
                ivec = idx_v[pl.ds(base + r0, L)]
                for j in range(L):
                    half = (lax.shift_right_logical(ivec[j], HSHIFT) & 1) * D_EMB
                    s = jnp.take(svec, jnp.full((L,), j, jnp.int32))
                    bag = (r0 + j) // BAG
                    for c in range(4):
                        ch = rowbuf[b, r0 + j, pl.ds(half + c * L, L)]
                        plsc.addupdate(acc_v.at[bag, pl.ds(c * L, L)], s * ch)
                return carry

            lax.fori_loop(0, GROWS // L, acc_body, 0)

            base_row = wid * bags_per_worker + g * GBAGS
            pltpu.sync_copy(acc_v, res_hbm.at[pl.ds(base_row, GBAGS)])

        issue(0, 0)
        issue(1, 1)

        def pair_body(i, carry):
            for b in range(2):
                g = i * 2 + b
                wait_group(b)
                compute(g, b)

                @pl.when(g + 2 < ngrp)
                def _():
                    issue(g + 2, b)
            return carry

        lax.fori_loop(0, ngrp // 2, pair_body, 0)

    return sc_gather


def kernel(input, output, input_table, output_table):
    batch = input.shape[0]
    gather = _make_gather(batch)
    n_in = input_table.shape[0]
    n_out = output_table.shape[0]

    in_tab = _make_tc_transpose(n_in)(input_table.T)
    out_tab = _make_tc_transpose(n_out)(output_table.T)

    in_res = gather(input.reshape(batch * BAG), in_tab)
    out_res = gather(output.reshape(batch * BAG), out_tab)
    return (in_res, out_res)


# speedup vs baseline: 4.4107x; 1.0849x over previous
import functools

import jax
import jax.numpy as jnp
from jax import lax
from jax.experimental import pallas as pl
from jax.experimental.pallas import tpu as pltpu
from jax.experimental.pallas import tpu_sc as plsc

D_EMB = 64
MAXN = 10.0
L = 16
NCORE = 2
NSUB = 16
NWORK = NCORE * NSUB
BAG = 50
GBAGS = 8
GROWS = GBAGS * BAG
NSTREAM = 5
SPG = GROWS // NSTREAM
RSQRT_MAGIC = 0x5F3759DF


BLK_COLS = 16384
VSHIFT = 14
HSHIFT = 13
VMASK = (1 << HSHIFT) - 1


def _make_tc_transpose(n, blk_cols=BLK_COLS):
    grid = (n + blk_cols - 1) // blk_cols
    half = blk_cols // 2

    def body(x_ref, o_ref):
        x = x_ref[...]
        eye = jnp.eye(D_EMB, dtype=jnp.float32)
        xt = lax.dot_general(x, eye, (((0,), (0,)), ((), ())),
                             precision=lax.Precision.DEFAULT,
                             preferred_element_type=jnp.float32)
        o_ref[...] = jnp.concatenate([xt[0:half], xt[half:blk_cols]], axis=1)

    return pl.pallas_call(
        body,
        grid=(grid,),
        in_specs=[pl.BlockSpec((D_EMB, blk_cols), lambda j: (0, j))],
        out_specs=pl.BlockSpec((half, 2 * D_EMB), lambda j: (j, 0)),
        out_shape=jax.ShapeDtypeStruct((grid * half, 2 * D_EMB), jnp.float32),
    )


def _rsqrt(x):
    i = lax.bitcast_convert_type(x, jnp.int32)
    y = lax.bitcast_convert_type(
        jnp.int32(RSQRT_MAGIC) - lax.shift_right_logical(i, 1), jnp.float32)
    for _ in range(2):
        y = y * (1.5 - 0.5 * x * y * y)
    return y


def _make_gather(batch):
    bags_per_worker = batch // NWORK
    ngrp = bags_per_worker // GBAGS
    rows_per_worker = ngrp * GROWS

    mesh = plsc.VectorSubcoreMesh(core_axis_name="c", subcore_axis_name="s")

    @functools.partial(
        pl.kernel,
        out_type=jax.ShapeDtypeStruct((batch, D_EMB), jnp.float32),
        mesh=mesh,
        scratch_types=[
            pltpu.VMEM((rows_per_worker,), jnp.int32),
            pltpu.VMEM((GROWS,), jnp.int32),
            pltpu.VMEM((GROWS,), jnp.int32),
            pltpu.VMEM((2, GROWS, 2 * D_EMB), jnp.float32),
            pltpu.VMEM((L * L,), jnp.float32),
            pltpu.VMEM((GROWS,), jnp.float32),
            pltpu.VMEM((GBAGS, D_EMB), jnp.float32),
            pltpu.SemaphoreType.DMA,
            pltpu.SemaphoreType.DMA,
        ],
        compiler_params=pltpu.CompilerParams(needs_layout_passes=False,
                                             use_tc_tiling_on_sc=True),
    )
    def sc_gather(idx_hbm, tab_hbm, res_hbm,
                  idx_v, sidx0, sidx1, rowbuf, tbuf, scale_v, acc_v,
                  sem0, sem1):
        wid = lax.axis_index("s") * NCORE + lax.axis_index("c")
        sems = (sem0, sem1)
        sidxs = (sidx0, sidx1)
        iota = lax.iota(jnp.int32, L)
        zero = jnp.zeros((L,), jnp.float32)

        pltpu.sync_copy(
            idx_hbm.at[pl.ds(wid * rows_per_worker, rows_per_worker)], idx_v)

        def issue(g, b):
            base = g * GROWS

            def shift_blk(k, carry):
                iv = idx_v[pl.ds(base + k * L, L)]
                vr = lax.shift_left(
                    lax.shift_right_logical(iv, VSHIFT), HSHIFT) | (
                        iv & VMASK)
                sidxs[b][pl.ds(k * L, L)] = vr
                return carry

            lax.fori_loop(0, GROWS // L, shift_blk, 0)
            for j in range(NSTREAM):
                pltpu.async_copy(
                    tab_hbm.at[sidxs[b].at[pl.ds(j * SPG, SPG)]],
                    rowbuf.at[b, pl.ds(j * SPG, SPG)],
                    sems[b])

        def wait_group(b):
            pltpu.make_async_copy(tab_hbm.at[pl.ds(0, GROWS)],
                                  rowbuf.at[b], sems[b]).wait()

        def compute(g, b):
            base = g * GROWS

            def blk_body(blk, carry):
                r0 = blk * L
                ivec = idx_v[pl.ds(base + r0, L)]
                for j in range(L):
                    half = (lax.shift_right_logical(ivec[j], HSHIFT) & 1) * D_EMB
                    n2v = None
                    for c in range(4):
                        ch = rowbuf[b, r0 + j, pl.ds(half + c * L, L)]
                        n2v = ch * ch if n2v is None else n2v + ch * ch
                    plsc.store_scatter(tbuf, [iota * L + j], n2v)
                n2 = tbuf[pl.ds(0, L)]
                for lrow in range(1, L):
                    n2 = n2 + tbuf[pl.ds(lrow * L, L)]
                scale = jnp.minimum(1.0, MAXN * _rsqrt(n2))
                scale_v[pl.ds(r0, L)] = scale
                return carry

            lax.fori_loop(0, GROWS // L, blk_body, 0)

            for bag in range(GBAGS):
                for c in range(4):
                    acc_v[bag, pl.ds(c * L, L)] = zero

            def acc_body(blk, carry):
                r0 = blk * L
                svec = scale_v[pl.ds(r0, L)]
                ivec = idx_v[pl.ds(base + r0, L)]
                for j in range(L):
                    half = (lax.shift_right_logical(ivec[j], HSHIFT) & 1) * D_EMB
                    s = jnp.take(svec, jnp.full((L,), j, jnp.int32))
                    bag = (r0 + j) // BAG
                    for c in range(4):
                        ch = rowbuf[b, r0 + j, pl.ds(half + c * L, L)]
                        plsc.addupdate(acc_v.at[bag, pl.ds(c * L, L)], s * ch)
                return carry

            lax.fori_loop(0, GROWS // L, acc_body, 0)

            base_row = wid * bags_per_worker + g * GBAGS
            pltpu.sync_copy(acc_v, res_hbm.at[pl.ds(base_row, GBAGS)])

        issue(0, 0)
        issue(1, 1)

        def pair_body(i, carry):
            for b in range(2):
                g = i * 2 + b
                wait_group(b)
                compute(g, b)

                @pl.when(g + 2 < ngrp)
                def _():
                    issue(g + 2, b)
            return carry

        lax.fori_loop(0, ngrp // 2, pair_body, 0)

    return sc_gather


def kernel(input, output, input_table, output_table):
    batch = input.shape[0]
    gather = _make_gather(batch)
    n_in = input_table.shape[0]
    n_out = output_table.shape[0]

    in_tab = _make_tc_transpose(n_in)(input_table.T)
    out_tab = _make_tc_transpose(n_out)(output_table.T)

    in_res = gather(input.reshape(batch * BAG), in_tab)
    out_res = gather(output.reshape(batch * BAG), out_tab)
    return (in_res, out_res)


# speedup vs baseline: 4.6286x; 1.0494x over previous
import functools

import jax
import jax.numpy as jnp
from jax import lax
from jax.experimental import pallas as pl
from jax.experimental.pallas import tpu as pltpu
from jax.experimental.pallas import tpu_sc as plsc

D_EMB = 64
MAXN = 10.0
L = 16
NCORE = 2
NSUB = 16
NWORK = NCORE * NSUB
BAG = 50
GBAGS = 8
GROWS = GBAGS * BAG
NSTREAM = 5
SPG = GROWS // NSTREAM
RSQRT_MAGIC = 0x5F3759DF


BLK_COLS = 32768
VSHIFT = 15
HSHIFT = 14
VMASK = (1 << HSHIFT) - 1


def _make_tc_transpose(n, blk_cols=BLK_COLS):
    grid = (n + blk_cols - 1) // blk_cols
    half = blk_cols // 2

    def body(x_ref, o_ref):
        x = x_ref[...]
        eye = jnp.eye(D_EMB, dtype=jnp.float32)
        xt = lax.dot_general(x, eye, (((0,), (0,)), ((), ())),
                             precision=lax.Precision.DEFAULT,
                             preferred_element_type=jnp.float32)
        o_ref[...] = jnp.concatenate([xt[0:half], xt[half:blk_cols]], axis=1)

    return pl.pallas_call(
        body,
        grid=(grid,),
        in_specs=[pl.BlockSpec((D_EMB, blk_cols), lambda j: (0, j))],
        out_specs=pl.BlockSpec((half, 2 * D_EMB), lambda j: (j, 0)),
        out_shape=jax.ShapeDtypeStruct((grid * half, 2 * D_EMB), jnp.float32),
    )


def _rsqrt(x):
    i = lax.bitcast_convert_type(x, jnp.int32)
    y = lax.bitcast_convert_type(
        jnp.int32(RSQRT_MAGIC) - lax.shift_right_logical(i, 1), jnp.float32)
    for _ in range(2):
        y = y * (1.5 - 0.5 * x * y * y)
    return y


def _make_gather(batch):
    bags_per_worker = batch // NWORK
    ngrp = bags_per_worker // GBAGS
    rows_per_worker = ngrp * GROWS

    mesh = plsc.VectorSubcoreMesh(core_axis_name="c", subcore_axis_name="s")

    @functools.partial(
        pl.kernel,
        out_type=jax.ShapeDtypeStruct((batch, D_EMB), jnp.float32),
        mesh=mesh,
        scratch_types=[
            pltpu.VMEM((rows_per_worker,), jnp.int32),
            pltpu.VMEM((GROWS,), jnp.int32),
            pltpu.VMEM((GROWS,), jnp.int32),
            pltpu.VMEM((2, GROWS, 2 * D_EMB), jnp.float32),
            pltpu.VMEM((L * L,), jnp.float32),
            pltpu.VMEM((GROWS,), jnp.float32),
            pltpu.VMEM((GBAGS, D_EMB), jnp.float32),
            pltpu.SemaphoreType.DMA,
            pltpu.SemaphoreType.DMA,
        ],
        compiler_params=pltpu.CompilerParams(needs_layout_passes=False,
                                             use_tc_tiling_on_sc=True),
    )
    def sc_gather(idx_hbm, tab_hbm, res_hbm,
                  idx_v, sidx0, sidx1, rowbuf, tbuf, scale_v, acc_v,
                  sem0, sem1):
        wid = lax.axis_index("s") * NCORE + lax.axis_index("c")
        sems = (sem0, sem1)
        sidxs = (sidx0, sidx1)
        iota = lax.iota(jnp.int32, L)
        zero = jnp.zeros((L,), jnp.float32)

        pltpu.sync_copy(
            idx_hbm.at[pl.ds(wid * rows_per_worker, rows_per_worker)], idx_v)

        def issue(g, b):
            base = g * GROWS

            def shift_blk(k, carry):
                iv = idx_v[pl.ds(base + k * L, L)]
                vr = lax.shift_left(
                    lax.shift_right_logical(iv, VSHIFT), HSHIFT) | (
                        iv & VMASK)
                sidxs[b][pl.ds(k * L, L)] = vr
                return carry

            lax.fori_loop(0, GROWS // L, shift_blk, 0)
            for j in range(NSTREAM):
                pltpu.async_copy(
                    tab_hbm.at[sidxs[b].at[pl.ds(j * SPG, SPG)]],
                    rowbuf.at[b, pl.ds(j * SPG, SPG)],
                    sems[b])

        def wait_group(b):
            pltpu.make_async_copy(tab_hbm.at[pl.ds(0, GROWS)],
                                  rowbuf.at[b], sems[b]).wait()

        def compute(g, b):
            base = g * GROWS

            def blk_body(blk, carry):
                r0 = blk * L
                ivec = idx_v[pl.ds(base + r0, L)]
                for j in range(L):
                    half = (lax.shift_right_logical(ivec[j], HSHIFT) & 1) * D_EMB
                    n2v = None
                    for c in range(4):
                        ch = rowbuf[b, r0 + j, pl.ds(half + c * L, L)]
                        n2v = ch * ch if n2v is None else n2v + ch * ch
                    plsc.store_scatter(tbuf, [iota * L + j], n2v)
                n2 = tbuf[pl.ds(0, L)]
                for lrow in range(1, L):
                    n2 = n2 + tbuf[pl.ds(lrow * L, L)]
                scale = jnp.minimum(1.0, MAXN * _rsqrt(n2))
                scale_v[pl.ds(r0, L)] = scale
                return carry

            lax.fori_loop(0, GROWS // L, blk_body, 0)

            for bag in range(GBAGS):
                for c in range(4):
                    acc_v[bag, pl.ds(c * L, L)] = zero

            def acc_body(blk, carry):
                r0 = blk * L
                svec = scale_v[pl.ds(r0, L)]
                ivec = idx_v[pl.ds(base + r0, L)]
                for j in range(L):
                    half = (lax.shift_right_logical(ivec[j], HSHIFT) & 1) * D_EMB
                    s = jnp.take(svec, jnp.full((L,), j, jnp.int32))
                    bag = (r0 + j) // BAG
                    for c in range(4):
                        ch = rowbuf[b, r0 + j, pl.ds(half + c * L, L)]
                        plsc.addupdate(acc_v.at[bag, pl.ds(c * L, L)], s * ch)
                return carry

            lax.fori_loop(0, GROWS // L, acc_body, 0)

            base_row = wid * bags_per_worker + g * GBAGS
            pltpu.sync_copy(acc_v, res_hbm.at[pl.ds(base_row, GBAGS)])

        issue(0, 0)
        issue(1, 1)

        def pair_body(i, carry):
            for b in range(2):
                g = i * 2 + b
                wait_group(b)
                compute(g, b)

                @pl.when(g + 2 < ngrp)
                def _():
                    issue(g + 2, b)
            return carry

        lax.fori_loop(0, ngrp // 2, pair_body, 0)

    return sc_gather


def kernel(input, output, input_table, output_table):
    batch = input.shape[0]
    gather = _make_gather(batch)
    n_in = input_table.shape[0]
    n_out = output_table.shape[0]

    in_tab = _make_tc_transpose(n_in)(input_table.T)
    out_tab = _make_tc_transpose(n_out)(output_table.T)

    in_res = gather(input.reshape(batch * BAG), in_tab)
    out_res = gather(output.reshape(batch * BAG), out_tab)
    return (in_res, out_res)
